# baseline (speedup 1.0000x reference)
import functools
import jax
import jax.numpy as jnp
from jax import lax
from jax.experimental import pallas as pl
from jax.experimental.pallas import tpu as pltpu
from jax.experimental.pallas import tpu_sc as plsc

_NE = 8192
_D = 256
_CB = 256
_BIG = 2**30
_NC = 2
_NS = 16
_NW = _NC * _NS


def _argmin_kernel(x_ref, e_ref, t_ref, idx_ref, loss_ref):
    b = pl.program_id(0)
    nb = pl.num_programs(0)
    xb = x_ref[0]
    xb2 = (xb * 2.0).astype(jnp.bfloat16)
    t = t_ref[0, 0]

    def chunk(c, carry):
        runval, runtrue, runidx = carry
        e = e_ref[pl.ds(c * _CB, _CB), :]
        mm = lax.dot_general(e, xb2, (((1,), (0,)), ((), ())),
                             preferred_element_type=jnp.float32)
        v = jnp.sum(e * e, axis=1)
        d = (t[None, :] - mm) + v[:, None]
        cmin = jnp.min(d, axis=0)
        iota = lax.broadcasted_iota(jnp.int32, (_CB, 1024), 0) + c * _CB
        cidx = jnp.min(jnp.where(d == cmin[None, :], iota, _BIG), axis=0)
        take = (cmin < runval) | ((cmin == runval) & (cidx < runidx))
        runidx = jnp.where(take, cidx, runidx)
        runtrue = jnp.where(take, cmin, runtrue)
        runval = jnp.where(take, cmin, runval)
        boundary = (c == 10) | (c == 21)
        rounded = runval.astype(jnp.bfloat16).astype(jnp.float32)
        runval = jnp.where(boundary, rounded, runval)
        return runval, runtrue, runidx

    runval = jnp.full((1024,), jnp.inf, jnp.float32)
    runtrue = jnp.full((1024,), jnp.inf, jnp.float32)
    runidx = jnp.zeros((1024,), jnp.int32)
    runval, runtrue, runidx = lax.fori_loop(
        0, _NE // _CB, chunk, (runval, runtrue, runidx))
    idx_ref[0, 0, :] = runidx

    part = jnp.sum(runtrue)
    prev = jnp.where(b == 0, 0.0, loss_ref[0, 0])
    acc = prev + part
    scale = 1.25 / (8 * 1024 * _D)
    loss_ref[0, 0] = jnp.where(b == nb - 1, acc * scale, acc)


def _gather_body(e_hbm, idx_hbm, out_hbm, idx_v, rows_v, sem):
    wid = lax.axis_index("s") * _NC + lax.axis_index("c")
    per_w = 8192 // _NW
    base = wid * per_w
    pltpu.sync_copy(idx_hbm.at[pl.ds(base, per_w)], idx_v)
    pltpu.async_copy(e_hbm.at[idx_v], rows_v, sem).wait()
    pltpu.sync_copy(rows_v, out_hbm.at[pl.ds(base, per_w)])


def _st_kernel(x_ref, q_ref, o_ref):
    xb = x_ref[0]
    qt = q_ref[:].T
    o_ref[0] = xb + (qt - xb)


def kernel(inputs, embedding_weight):
    B, C, H, W = inputs.shape
    HW = H * W
    x3 = inputs.reshape(B, C, HW)
    t = jnp.sum(inputs ** 2, axis=1).reshape(B, 1, HW)

    idx, loss = pl.pallas_call(
        _argmin_kernel,
        grid=(B,),
        in_specs=[pl.BlockSpec((1, C, HW), lambda b: (b, 0, 0)),
                  pl.BlockSpec((_NE, _D), lambda b: (0, 0)),
                  pl.BlockSpec((1, 1, HW), lambda b: (b, 0, 0))],
        out_specs=[pl.BlockSpec((1, 1, HW), lambda b: (b, 0, 0)),
                   pl.BlockSpec((1, 1), lambda b: (0, 0),
                                memory_space=pltpu.SMEM)],
        out_shape=[jax.ShapeDtypeStruct((B, 1, HW), jnp.int32),
                   jax.ShapeDtypeStruct((1, 1), jnp.float32)],
    )(x3, embedding_weight, t)

    idx_flat = idx.reshape(B * HW)

    mesh = plsc.VectorSubcoreMesh(core_axis_name="c", subcore_axis_name="s")
    per_w = (B * HW) // _NW
    gather = pl.kernel(
        _gather_body,
        mesh=mesh,
        out_type=jax.ShapeDtypeStruct((B * HW, _D), jnp.float32),
        scratch_types=[
            pltpu.VMEM((per_w,), jnp.int32),
            pltpu.VMEM((per_w, _D), jnp.float32),
            pltpu.SemaphoreType.DMA,
        ],
    )
    q_flat = gather(embedding_weight, idx_flat)

    out3 = pl.pallas_call(
        _st_kernel,
        grid=(B,),
        in_specs=[pl.BlockSpec((1, C, HW), lambda b: (b, 0, 0)),
                  pl.BlockSpec((HW, _D), lambda b: (b, 0))],
        out_specs=pl.BlockSpec((1, C, HW), lambda b: (b, 0, 0)),
        out_shape=jax.ShapeDtypeStruct((B, C, HW), jnp.float32),
    )(x3, q_flat)

    return out3.reshape(B, C, H, W), loss[0, 0]


# speedup vs baseline: 1.2860x; 1.2860x over previous
import functools
import jax
import jax.numpy as jnp
from jax import lax
from jax.experimental import pallas as pl
from jax.experimental.pallas import tpu as pltpu
from jax.experimental.pallas import tpu_sc as plsc

_NE = 8192
_D = 256
_CB = 256
_BIG = 2**30
_NC = 2
_NS = 16
_NW = _NC * _NS


def _argmin_kernel(x_ref, e_ref, t_ref, idx_ref, loss_ref):
    b = pl.program_id(0)
    nb = pl.num_programs(0)
    xb = x_ref[0]
    xb2 = (xb * 2.0).astype(jnp.bfloat16)
    t = t_ref[0, 0]

    spans = []
    bounds = (0, 2816, 5632, 8192)
    for wi in range(3):
        s = bounds[wi]
        while s < bounds[wi + 1]:
            size = min(512, bounds[wi + 1] - s)
            s += size
            spans.append((s - size, size, s == bounds[wi + 1] and wi < 2))

    runval = jnp.full((1024,), jnp.inf, jnp.float32)
    runtrue = jnp.full((1024,), jnp.inf, jnp.float32)
    runidx = jnp.zeros((1024,), jnp.int32)
    for start, size, boundary in spans:
        e = e_ref[pl.ds(start, size), :]
        mm = lax.dot_general(e, xb2, (((1,), (0,)), ((), ())),
                             preferred_element_type=jnp.float32)
        v = jnp.sum(e * e, axis=1)
        d = (t[None, :] - mm) + v[:, None]
        cmin = jnp.min(d, axis=0)
        iota = lax.broadcasted_iota(jnp.int32, (size, 1024), 0) + start
        cidx = jnp.min(jnp.where(d == cmin[None, :], iota, _BIG), axis=0)
        take = (cmin < runval) | ((cmin == runval) & (cidx < runidx))
        runidx = jnp.where(take, cidx, runidx)
        runtrue = jnp.where(take, cmin, runtrue)
        runval = jnp.where(take, cmin, runval)
        if boundary:
            runval = runval.astype(jnp.bfloat16).astype(jnp.float32)
    idx_ref[0, 0, :] = runidx

    part = jnp.sum(runtrue)
    prev = jnp.where(b == 0, 0.0, loss_ref[0, 0])
    acc = prev + part
    scale = 1.25 / (8 * 1024 * _D)
    loss_ref[0, 0] = jnp.where(b == nb - 1, acc * scale, acc)


def _gather_body(e_hbm, idx_hbm, out_hbm, idx_v, rows_v, sem):
    wid = lax.axis_index("s") * _NC + lax.axis_index("c")
    per_w = 8192 // _NW
    base = wid * per_w
    pltpu.sync_copy(idx_hbm.at[pl.ds(base, per_w)], idx_v)
    pltpu.async_copy(e_hbm.at[idx_v], rows_v, sem).wait()
    pltpu.sync_copy(rows_v, out_hbm.at[pl.ds(base, per_w)])


def _st_kernel(x_ref, q_ref, o_ref):
    xb = x_ref[0]
    qt = q_ref[:].T
    o_ref[0] = xb + (qt - xb)


def kernel(inputs, embedding_weight):
    B, C, H, W = inputs.shape
    HW = H * W
    x3 = inputs.reshape(B, C, HW)
    t = jnp.sum(inputs ** 2, axis=1).reshape(B, 1, HW)

    idx, loss = pl.pallas_call(
        _argmin_kernel,
        grid=(B,),
        in_specs=[pl.BlockSpec((1, C, HW), lambda b: (b, 0, 0)),
                  pl.BlockSpec((_NE, _D), lambda b: (0, 0)),
                  pl.BlockSpec((1, 1, HW), lambda b: (b, 0, 0))],
        out_specs=[pl.BlockSpec((1, 1, HW), lambda b: (b, 0, 0)),
                   pl.BlockSpec((1, 1), lambda b: (0, 0),
                                memory_space=pltpu.SMEM)],
        out_shape=[jax.ShapeDtypeStruct((B, 1, HW), jnp.int32),
                   jax.ShapeDtypeStruct((1, 1), jnp.float32)],
    )(x3, embedding_weight, t)

    idx_flat = idx.reshape(B * HW)

    mesh = plsc.VectorSubcoreMesh(core_axis_name="c", subcore_axis_name="s")
    per_w = (B * HW) // _NW
    gather = pl.kernel(
        _gather_body,
        mesh=mesh,
        out_type=jax.ShapeDtypeStruct((B * HW, _D), jnp.float32),
        scratch_types=[
            pltpu.VMEM((per_w,), jnp.int32),
            pltpu.VMEM((per_w, _D), jnp.float32),
            pltpu.SemaphoreType.DMA,
        ],
    )
    q_flat = gather(embedding_weight, idx_flat)

    out3 = pl.pallas_call(
        _st_kernel,
        grid=(B,),
        in_specs=[pl.BlockSpec((1, C, HW), lambda b: (b, 0, 0)),
                  pl.BlockSpec((HW, _D), lambda b: (b, 0))],
        out_specs=pl.BlockSpec((1, C, HW), lambda b: (b, 0, 0)),
        out_shape=jax.ShapeDtypeStruct((B, C, HW), jnp.float32),
    )(x3, q_flat)

    return out3.reshape(B, C, H, W), loss[0, 0]
